# baseline (speedup 1.0000x reference)
import functools

import jax
import jax.numpy as jnp
from jax import lax
from jax.experimental import pallas as pl
from jax.experimental.pallas import tpu as pltpu
from jax.experimental.pallas import tpu_sc as plsc

NC, NS, LANES = 2, 16, 16
NW = NC * NS


def _sc_mesh():
    return plsc.VectorSubcoreMesh(
        core_axis_name="c", subcore_axis_name="s", num_cores=NC, num_subcores=NS
    )


def _wid():
    return lax.axis_index("s") * NC + lax.axis_index("c")


def _scores_body(h_ref, w_ref, b_ref, scores_ref, hs_ref):
    hv = h_ref[...]
    w = w_ref[...]
    wt_full = lax.dot_general(hv, w, (((1,), (1,)), ((), ())))
    wt = wt_full[:, 0:1]
    s = jax.nn.sigmoid(wt + b_ref[0])
    scores_ref[...] = s
    hs_ref[...] = hv * s


def _scores_tc(h, W2, b1):
    N, D = h.shape
    return pl.pallas_call(
        _scores_body,
        in_specs=[
            pl.BlockSpec(memory_space=pltpu.MemorySpace.VMEM),
            pl.BlockSpec(memory_space=pltpu.MemorySpace.VMEM),
            pl.BlockSpec(memory_space=pltpu.MemorySpace.SMEM),
        ],
        out_shape=[
            jax.ShapeDtypeStruct((N, 1), jnp.float32),
            jax.ShapeDtypeStruct((N, D), jnp.float32),
        ],
    )(h, W2, b1)


_BR = 256


def _rank_body(sc_ref, sr_ref, rank_ref):
    i = pl.program_id(0)
    sc = sc_ref[...]
    sr = sr_ref[...]
    n = sr.shape[1]
    gt = (sr > sc).astype(jnp.float32)
    jrow = lax.broadcasted_iota(jnp.int32, (_BR, n), 1)
    irow = lax.broadcasted_iota(jnp.int32, (_BR, n), 0) + i * _BR
    tie = ((sr == sc) & (jrow < irow)).astype(jnp.float32)
    cnt = jnp.sum(gt + tie, axis=1, keepdims=True)
    rank_ref[...] = cnt.astype(jnp.int32)


def _rank_tc(s_col, s_row):
    N = s_col.shape[0]
    return pl.pallas_call(
        _rank_body,
        grid=(N // _BR,),
        in_specs=[
            pl.BlockSpec((_BR, 1), lambda i: (i, 0)),
            pl.BlockSpec((1, N), lambda i: (0, 0)),
        ],
        out_specs=pl.BlockSpec((_BR, 1), lambda i: (i, 0)),
        out_shape=jax.ShapeDtypeStruct((N, 1), jnp.int32),
    )(s_col, s_row)


def _select_sc(rank1, kk):
    N = rank1.shape[0]
    per = kk // NW

    @functools.partial(
        pl.kernel,
        out_type=jax.ShapeDtypeStruct((kk,), jnp.int32),
        mesh=_sc_mesh(),
        compiler_params=pltpu.CompilerParams(needs_layout_passes=False),
        scratch_types=[
            pltpu.VMEM((N,), jnp.int32),
            pltpu.VMEM((per,), jnp.int32),
        ],
    )
    def sel(rank_hbm, idx_hbm, rank_v, buf_v):
        lo = _wid() * per
        pltpu.sync_copy(rank_hbm, rank_v)

        def body(c, carry):
            r = rank_v[pl.ds(c * LANES, LANES)]
            iv = lax.iota(jnp.int32, LANES) + c * LANES
            m = (r >= lo) & (r < lo + per)
            rr = jnp.where(m, r - lo, 0)
            plsc.store_scatter(buf_v, [rr], iv, mask=m)
            return carry

        lax.fori_loop(0, N // LANES, body, 0)
        pltpu.sync_copy(buf_v, idx_hbm.at[pl.ds(lo, per)])

    return sel(rank1)


def _gather_sc(g, hs, idx):
    N = g.shape[0]
    D = hs.shape[1]
    kk = idx.shape[0]
    per = kk // NW
    chunks = per // LANES

    @functools.partial(
        pl.kernel,
        out_type=[
            jax.ShapeDtypeStruct((kk, N), jnp.float32),
            jax.ShapeDtypeStruct((kk, D), jnp.float32),
        ],
        mesh=_sc_mesh(),
        compiler_params=pltpu.CompilerParams(needs_layout_passes=False),
        scratch_types=[
            pltpu.VMEM((per,), jnp.int32),
            pltpu.VMEM((LANES, N), jnp.float32),
            pltpu.VMEM((per, D), jnp.float32),
            pltpu.SemaphoreType.DMA,
            pltpu.SemaphoreType.DMA,
        ],
    )
    def gat(g_hbm, hs_hbm, idx_hbm, a_hbm, nh_hbm, idx_v, grow_v, hrow_v, gsem, hsem):
        base = _wid() * per
        pltpu.sync_copy(idx_hbm.at[pl.ds(base, per)], idx_v)
        hcp = pltpu.async_copy(hs_hbm.at[idx_v], hrow_v, hsem)
        for t in range(chunks):
            ivr = idx_v[pl.ds(t * LANES, LANES)]
            pltpu.async_copy(g_hbm.at[ivr], grow_v, gsem).wait()
            pltpu.sync_copy(grow_v, a_hbm.at[pl.ds(base + t * LANES, LANES)])
        hcp.wait()
        pltpu.sync_copy(hrow_v, nh_hbm.at[pl.ds(base, per)])

    return gat(g, hs, idx)


_BJ = 256


def _mm_body(a_ref, g_ref, o_ref):
    acc = jnp.dot(a_ref[...], g_ref[...], preferred_element_type=jnp.float32)
    o_ref[...] = (jnp.transpose(acc) > 0.0).astype(jnp.bfloat16)


def _matmul_tc(A, g):
    kk, N = A.shape
    return pl.pallas_call(
        _mm_body,
        grid=(N // _BJ,),
        in_specs=[
            pl.BlockSpec((kk, N), lambda j: (0, 0)),
            pl.BlockSpec((N, _BJ), lambda j: (0, j)),
        ],
        out_specs=pl.BlockSpec((_BJ, kk), lambda j: (j, 0)),
        out_shape=jax.ShapeDtypeStruct((N, kk), jnp.bfloat16),
    )(A, g)


def _rowgather_sc(MT, idx):
    kk = idx.shape[0]
    w = MT.shape[1]
    per = kk // NW

    @functools.partial(
        pl.kernel,
        out_type=jax.ShapeDtypeStruct((kk, w), MT.dtype),
        mesh=_sc_mesh(),
        compiler_params=pltpu.CompilerParams(needs_layout_passes=False),
        scratch_types=[
            pltpu.VMEM((per,), jnp.int32),
            pltpu.VMEM((per, w), MT.dtype),
            pltpu.SemaphoreType.DMA,
        ],
    )
    def rg(mt_hbm, idx_hbm, out_hbm, idx_v, rows_v, sem):
        base = _wid() * per
        pltpu.sync_copy(idx_hbm.at[pl.ds(base, per)], idx_v)
        pltpu.async_copy(mt_hbm.at[idx_v], rows_v, sem).wait()
        pltpu.sync_copy(rows_v, out_hbm.at[pl.ds(base, per)])

    return rg(MT, idx)


_BT = 512


def _transpose_body(i_ref, o_ref):
    o_ref[...] = jnp.transpose(i_ref[...]).astype(jnp.float32)


def _transpose_tc(X):
    kk = X.shape[0]
    nb = kk // _BT
    return pl.pallas_call(
        _transpose_body,
        grid=(nb, nb),
        in_specs=[pl.BlockSpec((_BT, _BT), lambda i, j: (j, i))],
        out_specs=pl.BlockSpec((_BT, _BT), lambda i, j: (i, j)),
        out_shape=jax.ShapeDtypeStruct((kk, kk), jnp.float32),
    )(X)


def kernel(g, h, ep, W, b):
    N, D = h.shape
    kk = max(2, N // 2)
    Wp = jnp.pad(W, ((0, 127), (0, 0)))
    scores, hs = _scores_tc(h, Wp, b)
    rank = _rank_tc(scores, scores.reshape(1, N))
    idx = _select_sc(rank.reshape(N), kk)
    A, new_h = _gather_sc(g, hs, idx)
    MT = _matmul_tc(A, g)
    MT32 = lax.bitcast_convert_type(MT.reshape(N, kk // 2, 2), jnp.int32)
    un_gT32 = _rowgather_sc(MT32, idx)
    un_gT = lax.bitcast_convert_type(un_gT32, jnp.bfloat16).reshape(kk, kk)
    un_g = _transpose_tc(un_gT)
    return un_g, new_h, idx


# speedup vs baseline: 2.1171x; 2.1171x over previous
import functools

import jax
import jax.numpy as jnp
from jax import lax
from jax.experimental import pallas as pl
from jax.experimental.pallas import tpu as pltpu
from jax.experimental.pallas import tpu_sc as plsc

NC, NS, LANES = 2, 16, 16
NW = NC * NS


def _sc_mesh():
    return plsc.VectorSubcoreMesh(
        core_axis_name="c", subcore_axis_name="s", num_cores=NC, num_subcores=NS
    )


def _wid():
    return lax.axis_index("s") * NC + lax.axis_index("c")


def _scores_body(h_ref, w_ref, b_ref, scores_ref, hs_ref):
    hv = h_ref[...]
    w = w_ref[...]
    wt_full = lax.dot_general(hv, w, (((1,), (1,)), ((), ())))
    wt = wt_full[:, 0:1]
    s = jax.nn.sigmoid(wt + b_ref[0])
    scores_ref[...] = s
    hs_ref[...] = hv * s


def _scores_tc(h, W2, b1):
    N, D = h.shape
    return pl.pallas_call(
        _scores_body,
        in_specs=[
            pl.BlockSpec(memory_space=pltpu.MemorySpace.VMEM),
            pl.BlockSpec(memory_space=pltpu.MemorySpace.VMEM),
            pl.BlockSpec(memory_space=pltpu.MemorySpace.SMEM),
        ],
        out_shape=[
            jax.ShapeDtypeStruct((N, 1), jnp.float32),
            jax.ShapeDtypeStruct((N, D), jnp.float32),
        ],
    )(h, W2, b1)


_BR = 256


def _rank_body(sc_ref, sr_ref, rank_ref):
    i = pl.program_id(0)
    sc = sc_ref[...]
    sr = sr_ref[...]
    n = sr.shape[1]
    gt = (sr > sc).astype(jnp.float32)
    jrow = lax.broadcasted_iota(jnp.int32, (_BR, n), 1)
    irow = lax.broadcasted_iota(jnp.int32, (_BR, n), 0) + i * _BR
    tie = ((sr == sc) & (jrow < irow)).astype(jnp.float32)
    cnt = jnp.sum(gt + tie, axis=1, keepdims=True)
    rank_ref[...] = cnt.astype(jnp.int32)


def _rank_tc(s_col, s_row):
    N = s_col.shape[0]
    return pl.pallas_call(
        _rank_body,
        grid=(N // _BR,),
        in_specs=[
            pl.BlockSpec((_BR, 1), lambda i: (i, 0)),
            pl.BlockSpec((1, N), lambda i: (0, 0)),
        ],
        out_specs=pl.BlockSpec((_BR, 1), lambda i: (i, 0)),
        out_shape=jax.ShapeDtypeStruct((N, 1), jnp.int32),
    )(s_col, s_row)


def _select_sc(rank1, kk):
    N = rank1.shape[0]
    per = kk // NW

    @functools.partial(
        pl.kernel,
        out_type=jax.ShapeDtypeStruct((kk,), jnp.int32),
        mesh=_sc_mesh(),
        compiler_params=pltpu.CompilerParams(needs_layout_passes=False),
        scratch_types=[
            pltpu.VMEM((N,), jnp.int32),
            pltpu.VMEM((per,), jnp.int32),
        ],
    )
    def sel(rank_hbm, idx_hbm, rank_v, buf_v):
        lo = _wid() * per
        pltpu.sync_copy(rank_hbm, rank_v)

        def body(c, carry):
            r = rank_v[pl.ds(c * LANES, LANES)]
            iv = lax.iota(jnp.int32, LANES) + c * LANES
            m = (r >= lo) & (r < lo + per)
            rr = jnp.where(m, r - lo, 0)
            plsc.store_scatter(buf_v, [rr], iv, mask=m)
            return carry

        lax.fori_loop(0, N // LANES, body, 0)
        pltpu.sync_copy(buf_v, idx_hbm.at[pl.ds(lo, per)])

    return sel(rank1)


def _gather_sc(g, hs, idx):
    N = g.shape[0]
    D = hs.shape[1]
    kk = idx.shape[0]
    per = kk // NW
    ch = 8
    nch = per // ch

    @functools.partial(
        pl.kernel,
        out_type=[
            jax.ShapeDtypeStruct((kk, N), jnp.float32),
            jax.ShapeDtypeStruct((kk, D), jnp.float32),
        ],
        mesh=_sc_mesh(),
        compiler_params=pltpu.CompilerParams(needs_layout_passes=False),
        scratch_types=[
            pltpu.VMEM((per,), jnp.int32),
            pltpu.VMEM((2, ch, N), jnp.float32),
            pltpu.VMEM((per, D), jnp.float32),
            pltpu.SemaphoreType.DMA,
            pltpu.SemaphoreType.DMA,
            pltpu.SemaphoreType.DMA,
        ],
    )
    def gat(
        g_hbm, hs_hbm, idx_hbm, a_hbm, nh_hbm, idx_v, grow_v, hrow_v, s0, s1, hsem
    ):
        base = _wid() * per
        pltpu.sync_copy(idx_hbm.at[pl.ds(base, per)], idx_v)
        hcp = pltpu.async_copy(hs_hbm.at[idx_v], hrow_v, hsem)
        sems = (s0, s1)
        cps = [None] * nch
        cps[0] = pltpu.async_copy(g_hbm.at[idx_v.at[pl.ds(0, ch)]], grow_v.at[0], sems[0])
        for t in range(nch):
            if t + 1 < nch:
                cps[t + 1] = pltpu.async_copy(
                    g_hbm.at[idx_v.at[pl.ds((t + 1) * ch, ch)]],
                    grow_v.at[(t + 1) % 2],
                    sems[(t + 1) % 2],
                )
            cps[t].wait()
            pltpu.sync_copy(grow_v.at[t % 2], a_hbm.at[pl.ds(base + t * ch, ch)])
        hcp.wait()
        pltpu.sync_copy(hrow_v, nh_hbm.at[pl.ds(base, per)])

    return gat(g, hs, idx)


_BJ = 256


def _mm_body(a_ref, g_ref, o_ref):
    acc = jnp.dot(a_ref[...], g_ref[...], preferred_element_type=jnp.float32)
    o_ref[...] = (jnp.transpose(acc) > 0.0).astype(jnp.float32)


def _matmul_tc(A, g):
    kk, N = A.shape
    return pl.pallas_call(
        _mm_body,
        grid=(N // _BJ,),
        in_specs=[
            pl.BlockSpec((kk, N), lambda j: (0, 0)),
            pl.BlockSpec((N, _BJ), lambda j: (0, j)),
        ],
        out_specs=pl.BlockSpec((_BJ, kk), lambda j: (j, 0)),
        out_shape=jax.ShapeDtypeStruct((N, kk), jnp.float32),
    )(A, g)


def _rowgather_sc(MT, idx):
    kk = idx.shape[0]
    w = MT.shape[1]
    per = kk // NW
    ch = LANES
    nch = per // ch

    @functools.partial(
        pl.kernel,
        out_type=jax.ShapeDtypeStruct((kk, w), MT.dtype),
        mesh=_sc_mesh(),
        compiler_params=pltpu.CompilerParams(needs_layout_passes=False),
        scratch_types=[
            pltpu.VMEM((per,), jnp.int32),
            pltpu.VMEM((2, ch, w), MT.dtype),
            pltpu.SemaphoreType.DMA,
            pltpu.SemaphoreType.DMA,
        ],
    )
    def rg(mt_hbm, idx_hbm, out_hbm, idx_v, rows_v, sem0, sem1):
        base = _wid() * per
        pltpu.sync_copy(idx_hbm.at[pl.ds(base, per)], idx_v)
        sems = (sem0, sem1)
        cps = [None] * nch
        cps[0] = pltpu.async_copy(
            mt_hbm.at[idx_v.at[pl.ds(0, ch)]], rows_v.at[0], sems[0]
        )
        for t in range(nch):
            if t + 1 < nch:
                cps[t + 1] = pltpu.async_copy(
                    mt_hbm.at[idx_v.at[pl.ds((t + 1) * ch, ch)]],
                    rows_v.at[(t + 1) % 2],
                    sems[(t + 1) % 2],
                )
            cps[t].wait()
            pltpu.sync_copy(rows_v.at[t % 2], out_hbm.at[pl.ds(base + t * ch, ch)])

    return rg(MT, idx)


_BT = 512


def _transpose_body(i_ref, o_ref):
    o_ref[...] = jnp.transpose(i_ref[...]).astype(jnp.float32)


def _transpose_tc(X):
    kk = X.shape[0]
    nb = kk // _BT
    return pl.pallas_call(
        _transpose_body,
        grid=(nb, nb),
        in_specs=[pl.BlockSpec((_BT, _BT), lambda i, j: (j, i))],
        out_specs=pl.BlockSpec((_BT, _BT), lambda i, j: (i, j)),
        out_shape=jax.ShapeDtypeStruct((kk, kk), jnp.float32),
    )(X)


def kernel(g, h, ep, W, b):
    N, D = h.shape
    kk = max(2, N // 2)
    Wp = jnp.pad(W, ((0, 127), (0, 0)))
    scores, hs = _scores_tc(h, Wp, b)
    rank = _rank_tc(scores, scores.reshape(1, N))
    idx = _select_sc(rank.reshape(N), kk)
    A, new_h = _gather_sc(g, hs, idx)
    MT = _matmul_tc(A, g)
    un_gT = _rowgather_sc(MT, idx)
    un_g = _transpose_tc(un_gT)
    return un_g, new_h, idx


# speedup vs baseline: 2.2957x; 1.0844x over previous
import functools

import jax
import jax.numpy as jnp
from jax import lax
from jax.experimental import pallas as pl
from jax.experimental.pallas import tpu as pltpu
from jax.experimental.pallas import tpu_sc as plsc

NC, NS, LANES = 2, 16, 16
NW = NC * NS


def _sc_mesh():
    return plsc.VectorSubcoreMesh(
        core_axis_name="c", subcore_axis_name="s", num_cores=NC, num_subcores=NS
    )


def _wid():
    return lax.axis_index("s") * NC + lax.axis_index("c")


_BR = 256


def _scorank_body(h_ref, w_ref, b_ref, rank_ref, hs_ref):
    hv = h_ref[...]
    w = w_ref[...]
    wt_full = lax.dot_general(hv, w, (((1,), (1,)), ((), ())))
    wt = wt_full[:, 0:1]
    s = jax.nn.sigmoid(wt + b_ref[0])
    hs_ref[...] = hv * s
    n = s.shape[0]
    sr = jnp.transpose(s)
    for bi in range(n // _BR):
        lo = bi * _BR
        sc = s[lo : lo + _BR, :]
        cnt = jnp.zeros((_BR, 1), jnp.float32)
        if lo > 0:
            pre = sr[:, :lo]
            cnt += jnp.sum((pre >= sc).astype(jnp.float32), axis=1, keepdims=True)
        if lo + _BR < n:
            post = sr[:, lo + _BR :]
            cnt += jnp.sum((post > sc).astype(jnp.float32), axis=1, keepdims=True)
        diag = sr[:, lo : lo + _BR]
        jrow = lax.broadcasted_iota(jnp.int32, (_BR, _BR), 1)
        irow = lax.broadcasted_iota(jnp.int32, (_BR, _BR), 0)
        dcnt = jnp.where(jrow < irow, (diag >= sc).astype(jnp.float32), 0.0) + (
            jnp.where(jrow >= irow, (diag > sc).astype(jnp.float32), 0.0)
        )
        cnt += jnp.sum(dcnt, axis=1, keepdims=True)
        rank_ref[lo : lo + _BR, :] = cnt.astype(jnp.int32)


def _scorank_tc(h, W2, b1):
    N, D = h.shape
    return pl.pallas_call(
        _scorank_body,
        in_specs=[
            pl.BlockSpec(memory_space=pltpu.MemorySpace.VMEM),
            pl.BlockSpec(memory_space=pltpu.MemorySpace.VMEM),
            pl.BlockSpec(memory_space=pltpu.MemorySpace.SMEM),
        ],
        out_shape=[
            jax.ShapeDtypeStruct((N, 1), jnp.int32),
            jax.ShapeDtypeStruct((N, D), jnp.float32),
        ],
    )(h, W2, b1)


def _select_sc(rank1, kk):
    N = rank1.shape[0]
    per = kk // NW

    @functools.partial(
        pl.kernel,
        out_type=jax.ShapeDtypeStruct((kk,), jnp.int32),
        mesh=_sc_mesh(),
        compiler_params=pltpu.CompilerParams(needs_layout_passes=False),
        scratch_types=[
            pltpu.VMEM((N,), jnp.int32),
            pltpu.VMEM((per,), jnp.int32),
        ],
    )
    def sel(rank_hbm, idx_hbm, rank_v, buf_v):
        lo = _wid() * per
        pltpu.sync_copy(rank_hbm, rank_v)

        def body(c, carry):
            r = rank_v[pl.ds(c * LANES, LANES)]
            iv = lax.iota(jnp.int32, LANES) + c * LANES
            m = (r >= lo) & (r < lo + per)
            rr = jnp.where(m, r - lo, 0)
            plsc.store_scatter(buf_v, [rr], iv, mask=m)
            return carry

        lax.fori_loop(0, N // LANES, body, 0)
        pltpu.sync_copy(buf_v, idx_hbm.at[pl.ds(lo, per)])

    return sel(rank1)


def _gather_sc(g, hs, idx):
    N = g.shape[0]
    D = hs.shape[1]
    kk = idx.shape[0]
    per = kk // NW
    ch = 8
    nch = per // ch

    @functools.partial(
        pl.kernel,
        out_type=[
            jax.ShapeDtypeStruct((kk, N), jnp.float32),
            jax.ShapeDtypeStruct((kk, D), jnp.float32),
        ],
        mesh=_sc_mesh(),
        compiler_params=pltpu.CompilerParams(needs_layout_passes=False),
        scratch_types=[
            pltpu.VMEM((per,), jnp.int32),
            pltpu.VMEM((2, ch, N), jnp.float32),
            pltpu.VMEM((per, D), jnp.float32),
            pltpu.SemaphoreType.DMA,
            pltpu.SemaphoreType.DMA,
            pltpu.SemaphoreType.DMA,
        ],
    )
    def gat(
        g_hbm, hs_hbm, idx_hbm, a_hbm, nh_hbm, idx_v, grow_v, hrow_v, s0, s1, hsem
    ):
        base = _wid() * per
        pltpu.sync_copy(idx_hbm.at[pl.ds(base, per)], idx_v)
        hcp = pltpu.async_copy(hs_hbm.at[idx_v], hrow_v, hsem)
        sems = (s0, s1)
        cps = [None] * nch
        cps[0] = pltpu.async_copy(g_hbm.at[idx_v.at[pl.ds(0, ch)]], grow_v.at[0], sems[0])
        for t in range(nch):
            if t + 1 < nch:
                cps[t + 1] = pltpu.async_copy(
                    g_hbm.at[idx_v.at[pl.ds((t + 1) * ch, ch)]],
                    grow_v.at[(t + 1) % 2],
                    sems[(t + 1) % 2],
                )
            cps[t].wait()
            pltpu.sync_copy(grow_v.at[t % 2], a_hbm.at[pl.ds(base + t * ch, ch)])
        hcp.wait()
        pltpu.sync_copy(hrow_v, nh_hbm.at[pl.ds(base, per)])

    return gat(g, hs, idx)


_BJ = 256


def _mm_body(a_ref, g_ref, o_ref):
    acc = jnp.dot(a_ref[...], g_ref[...], preferred_element_type=jnp.float32)
    o_ref[...] = (jnp.transpose(acc) > 0.0).astype(jnp.float32)


def _matmul_tc(A, g):
    kk, N = A.shape
    return pl.pallas_call(
        _mm_body,
        grid=(N // _BJ,),
        in_specs=[
            pl.BlockSpec((kk, N), lambda j: (0, 0)),
            pl.BlockSpec((N, _BJ), lambda j: (0, j)),
        ],
        out_specs=pl.BlockSpec((_BJ, kk), lambda j: (j, 0)),
        out_shape=jax.ShapeDtypeStruct((N, kk), jnp.float32),
    )(A, g)


def _rowgather_sc(MT, idx):
    kk = idx.shape[0]
    w = MT.shape[1]
    per = kk // NW
    ch = LANES
    nch = per // ch

    @functools.partial(
        pl.kernel,
        out_type=jax.ShapeDtypeStruct((kk, w), MT.dtype),
        mesh=_sc_mesh(),
        compiler_params=pltpu.CompilerParams(needs_layout_passes=False),
        scratch_types=[
            pltpu.VMEM((per,), jnp.int32),
            pltpu.VMEM((2, ch, w), MT.dtype),
            pltpu.SemaphoreType.DMA,
            pltpu.SemaphoreType.DMA,
        ],
    )
    def rg(mt_hbm, idx_hbm, out_hbm, idx_v, rows_v, sem0, sem1):
        base = _wid() * per
        pltpu.sync_copy(idx_hbm.at[pl.ds(base, per)], idx_v)
        sems = (sem0, sem1)
        cps = [None] * nch
        cps[0] = pltpu.async_copy(
            mt_hbm.at[idx_v.at[pl.ds(0, ch)]], rows_v.at[0], sems[0]
        )
        for t in range(nch):
            if t + 1 < nch:
                cps[t + 1] = pltpu.async_copy(
                    mt_hbm.at[idx_v.at[pl.ds((t + 1) * ch, ch)]],
                    rows_v.at[(t + 1) % 2],
                    sems[(t + 1) % 2],
                )
            cps[t].wait()
            pltpu.sync_copy(rows_v.at[t % 2], out_hbm.at[pl.ds(base + t * ch, ch)])

    return rg(MT, idx)


_BT = 512


def _transpose_body(i_ref, o_ref):
    o_ref[...] = jnp.transpose(i_ref[...]).astype(jnp.float32)


def _transpose_tc(X):
    kk = X.shape[0]
    nb = kk // _BT
    return pl.pallas_call(
        _transpose_body,
        grid=(nb, nb),
        in_specs=[pl.BlockSpec((_BT, _BT), lambda i, j: (j, i))],
        out_specs=pl.BlockSpec((_BT, _BT), lambda i, j: (i, j)),
        out_shape=jax.ShapeDtypeStruct((kk, kk), jnp.float32),
    )(X)


def kernel(g, h, ep, W, b):
    N, D = h.shape
    kk = max(2, N // 2)
    Wp = jnp.pad(W, ((0, 127), (0, 0)))
    rank, hs = _scorank_tc(h, Wp, b)
    idx = _select_sc(rank.reshape(N), kk)
    A, new_h = _gather_sc(g, hs, idx)
    MT = _matmul_tc(A, g)
    un_gT = _rowgather_sc(MT, idx)
    un_g = _transpose_tc(un_gT)
    return un_g, new_h, idx


# speedup vs baseline: 2.3575x; 1.0269x over previous
import functools

import jax
import jax.numpy as jnp
from jax import lax
from jax.experimental import pallas as pl
from jax.experimental.pallas import tpu as pltpu
from jax.experimental.pallas import tpu_sc as plsc

NC, NS, LANES = 2, 16, 16
NW = NC * NS


def _sc_mesh():
    return plsc.VectorSubcoreMesh(
        core_axis_name="c", subcore_axis_name="s", num_cores=NC, num_subcores=NS
    )


def _wid():
    return lax.axis_index("s") * NC + lax.axis_index("c")


_BR = 256


def _scorank_body(h_ref, w_ref, b_ref, rank_ref, hs_ref):
    hv = h_ref[...]
    w = w_ref[...]
    wt_full = lax.dot_general(hv, w, (((1,), (1,)), ((), ())))
    wt = wt_full[:, 0:1]
    s = jax.nn.sigmoid(wt + b_ref[0])
    hs_ref[...] = hv * s
    n = s.shape[0]
    sr = jnp.transpose(s)
    for bi in range(n // _BR):
        lo = bi * _BR
        sc = s[lo : lo + _BR, :]
        cnt = jnp.zeros((_BR, 1), jnp.float32)
        if lo > 0:
            pre = sr[:, :lo]
            cnt += jnp.sum((pre >= sc).astype(jnp.float32), axis=1, keepdims=True)
        if lo + _BR < n:
            post = sr[:, lo + _BR :]
            cnt += jnp.sum((post > sc).astype(jnp.float32), axis=1, keepdims=True)
        diag = sr[:, lo : lo + _BR]
        jrow = lax.broadcasted_iota(jnp.int32, (_BR, _BR), 1)
        irow = lax.broadcasted_iota(jnp.int32, (_BR, _BR), 0)
        dcnt = jnp.where(jrow < irow, (diag >= sc).astype(jnp.float32), 0.0) + (
            jnp.where(jrow >= irow, (diag > sc).astype(jnp.float32), 0.0)
        )
        cnt += jnp.sum(dcnt, axis=1, keepdims=True)
        rank_ref[lo : lo + _BR, :] = cnt.astype(jnp.int32)


def _scorank_tc(h, W2, b1):
    N, D = h.shape
    return pl.pallas_call(
        _scorank_body,
        in_specs=[
            pl.BlockSpec(memory_space=pltpu.MemorySpace.VMEM),
            pl.BlockSpec(memory_space=pltpu.MemorySpace.VMEM),
            pl.BlockSpec(memory_space=pltpu.MemorySpace.SMEM),
        ],
        out_shape=[
            jax.ShapeDtypeStruct((N, 1), jnp.int32),
            jax.ShapeDtypeStruct((N, D), jnp.float32),
        ],
    )(h, W2, b1)


def _selgather_sc(rank1, g, hs, kk):
    N = g.shape[0]
    D = hs.shape[1]
    per = kk // NW
    ch = 8
    nch = per // ch

    @functools.partial(
        pl.kernel,
        out_type=[
            jax.ShapeDtypeStruct((kk,), jnp.int32),
            jax.ShapeDtypeStruct((kk, N), jnp.float32),
            jax.ShapeDtypeStruct((kk, D), jnp.float32),
        ],
        mesh=_sc_mesh(),
        compiler_params=pltpu.CompilerParams(needs_layout_passes=False),
        scratch_types=[
            pltpu.VMEM((N,), jnp.int32),
            pltpu.VMEM((per,), jnp.int32),
            pltpu.VMEM((2, ch, N), jnp.float32),
            pltpu.VMEM((per, D), jnp.float32),
            pltpu.SemaphoreType.DMA,
            pltpu.SemaphoreType.DMA,
            pltpu.SemaphoreType.DMA,
        ],
    )
    def gat(
        rank_hbm, g_hbm, hs_hbm, idx_hbm, a_hbm, nh_hbm,
        rank_v, idx_v, grow_v, hrow_v, s0, s1, hsem,
    ):
        base = _wid() * per
        pltpu.sync_copy(rank_hbm, rank_v)

        def body(c, carry):
            r = rank_v[pl.ds(c * LANES, LANES)]
            iv = lax.iota(jnp.int32, LANES) + c * LANES
            m = (r >= base) & (r < base + per)
            rr = jnp.where(m, r - base, 0)
            plsc.store_scatter(idx_v, [rr], iv, mask=m)
            return carry

        lax.fori_loop(0, N // LANES, body, 0)
        pltpu.sync_copy(idx_v, idx_hbm.at[pl.ds(base, per)])

        hcp = pltpu.async_copy(hs_hbm.at[idx_v], hrow_v, hsem)
        sems = (s0, s1)
        cps = [None] * nch
        cps[0] = pltpu.async_copy(g_hbm.at[idx_v.at[pl.ds(0, ch)]], grow_v.at[0], sems[0])
        for t in range(nch):
            if t + 1 < nch:
                cps[t + 1] = pltpu.async_copy(
                    g_hbm.at[idx_v.at[pl.ds((t + 1) * ch, ch)]],
                    grow_v.at[(t + 1) % 2],
                    sems[(t + 1) % 2],
                )
            cps[t].wait()
            pltpu.sync_copy(grow_v.at[t % 2], a_hbm.at[pl.ds(base + t * ch, ch)])
        hcp.wait()
        pltpu.sync_copy(hrow_v, nh_hbm.at[pl.ds(base, per)])

    return gat(rank1, g, hs)


_BJ = 256


def _mm_body(a_ref, g_ref, o_ref):
    acc = jnp.dot(a_ref[...], g_ref[...], preferred_element_type=jnp.float32)
    o_ref[...] = (jnp.transpose(acc) > 0.0).astype(jnp.float32)


def _matmul_tc(A, g):
    kk, N = A.shape
    return pl.pallas_call(
        _mm_body,
        grid=(N // _BJ,),
        in_specs=[
            pl.BlockSpec((kk, N), lambda j: (0, 0)),
            pl.BlockSpec((N, _BJ), lambda j: (0, j)),
        ],
        out_specs=pl.BlockSpec((_BJ, kk), lambda j: (j, 0)),
        out_shape=jax.ShapeDtypeStruct((N, kk), jnp.float32),
    )(A, g)


def _rowgather_sc(MT, idx):
    kk = idx.shape[0]
    w = MT.shape[1]
    per = kk // NW
    ch = LANES
    nch = per // ch

    @functools.partial(
        pl.kernel,
        out_type=jax.ShapeDtypeStruct((kk, w), MT.dtype),
        mesh=_sc_mesh(),
        compiler_params=pltpu.CompilerParams(needs_layout_passes=False),
        scratch_types=[
            pltpu.VMEM((per,), jnp.int32),
            pltpu.VMEM((2, ch, w), MT.dtype),
            pltpu.SemaphoreType.DMA,
            pltpu.SemaphoreType.DMA,
        ],
    )
    def rg(mt_hbm, idx_hbm, out_hbm, idx_v, rows_v, sem0, sem1):
        base = _wid() * per
        pltpu.sync_copy(idx_hbm.at[pl.ds(base, per)], idx_v)
        sems = (sem0, sem1)
        cps = [None] * nch
        cps[0] = pltpu.async_copy(
            mt_hbm.at[idx_v.at[pl.ds(0, ch)]], rows_v.at[0], sems[0]
        )
        for t in range(nch):
            if t + 1 < nch:
                cps[t + 1] = pltpu.async_copy(
                    mt_hbm.at[idx_v.at[pl.ds((t + 1) * ch, ch)]],
                    rows_v.at[(t + 1) % 2],
                    sems[(t + 1) % 2],
                )
            cps[t].wait()
            pltpu.sync_copy(rows_v.at[t % 2], out_hbm.at[pl.ds(base + t * ch, ch)])

    return rg(MT, idx)


_BT = 512


def _transpose_body(i_ref, o_ref):
    o_ref[...] = jnp.transpose(i_ref[...]).astype(jnp.float32)


def _transpose_tc(X):
    kk = X.shape[0]
    nb = kk // _BT
    return pl.pallas_call(
        _transpose_body,
        grid=(nb, nb),
        in_specs=[pl.BlockSpec((_BT, _BT), lambda i, j: (j, i))],
        out_specs=pl.BlockSpec((_BT, _BT), lambda i, j: (i, j)),
        out_shape=jax.ShapeDtypeStruct((kk, kk), jnp.float32),
    )(X)


def kernel(g, h, ep, W, b):
    N, D = h.shape
    kk = max(2, N // 2)
    Wp = jnp.pad(W, ((0, 127), (0, 0)))
    rank, hs = _scorank_tc(h, Wp, b)
    idx, A, new_h = _selgather_sc(rank.reshape(N), g, hs, kk)
    MT = _matmul_tc(A, g)
    un_gT = _rowgather_sc(MT, idx)
    un_g = _transpose_tc(un_gT)
    return un_g, new_h, idx
